# baseline (speedup 1.0000x reference)
import jax
import jax.numpy as jnp
from jax.experimental import pallas as pl
from jax.experimental.pallas import tpu as pltpu

_CHUNK = 2000
_NCHUNKS = 500
_NBUF = 20
_NGROUPS = _NCHUNKS // _NBUF
_HALF = _NBUF // 2


def _copy_body(in_hbm, out_hbm, buf, sem_in, sem_out):
    def in_copy(k, b):
        base = pl.multiple_of(k * _CHUNK, 8)
        return pltpu.make_async_copy(
            in_hbm.at[pl.ds(base, _CHUNK), :], buf.at[b], sem_in.at[b]
        )

    def out_copy(k, b):
        base = pl.multiple_of(k * _CHUNK, 8)
        return pltpu.make_async_copy(
            buf.at[b], out_hbm.at[pl.ds(base, _CHUNK), :], sem_out.at[b]
        )

    for b in range(_HALF):
        in_copy(b, b).start()

    def group(g, carry):
        for t in range(_NBUF):
            k = g * _NBUF + t
            in_copy(k, t).wait()
            out_copy(k, t).start()
            bp = (t + _HALF) % _NBUF
            kp = k + _HALF

            @pl.when(k >= _HALF)
            def _drain_prev_out():
                out_copy(kp - _NBUF, bp).wait()

            @pl.when(kp < _NCHUNKS)
            def _prefetch():
                in_copy(kp, bp).start()

        return carry

    jax.lax.fori_loop(0, _NGROUPS, group, 0)

    for b in range(_HALF, _NBUF):
        out_copy(_NCHUNKS - _NBUF + b, b).wait()


def kernel(embeds):
    rows, cols = embeds.shape
    return pl.pallas_call(
        _copy_body,
        in_specs=[pl.BlockSpec(memory_space=pl.ANY)],
        out_specs=pl.BlockSpec(memory_space=pl.ANY),
        out_shape=jax.ShapeDtypeStruct((rows, cols), embeds.dtype),
        scratch_shapes=[
            pltpu.VMEM((_NBUF, _CHUNK, 64), jnp.float32),
            pltpu.SemaphoreType.DMA((_NBUF,)),
            pltpu.SemaphoreType.DMA((_NBUF,)),
        ],
    )(embeds)
